# baseline (speedup 1.0000x reference)
import functools

import jax
import jax.numpy as jnp
from jax import lax
from jax.experimental import pallas as pl
from jax.experimental.pallas import tpu as pltpu
from jax.experimental.pallas import tpu_sc as plsc

DIM = 64
BATCH2 = 32768
NC, NS, LANES = 2, 16, 16
NW = NC * NS
ROWS_PER_W = BATCH2 // NW
CHUNK = 128
NCHUNK = ROWS_PER_W // CHUNK

_LN2 = 0.6931471805599453


def _softplus16(x):
    t = 1.0 + jnp.exp(x)
    i = lax.bitcast_convert_type(t, jnp.int32)
    e = lax.shift_right_arithmetic(i, 23) - 127
    m = lax.bitcast_convert_type(
        lax.bitwise_or(lax.bitwise_and(i, 0x007FFFFF), 0x3F800000),
        jnp.float32)
    z = (m - 1.0) / (m + 1.0)
    z2 = z * z
    p = z * (2.0 + z2 * (2.0 / 3.0 + z2 * (2.0 / 5.0 + z2 * (2.0 / 7.0))))
    ln_t = e.astype(jnp.float32) * _LN2 + p
    return jnp.where(x > 20.0, x, ln_t)


def _sc_body(idx_hbm, u_hbm, v_hbm, out_hbm, idx_v, bufu, bufv, gsem, osem):
    wid = lax.axis_index("s") * NC + lax.axis_index("c")
    pltpu.sync_copy(idx_hbm.at[wid], idx_v)
    base = wid * ROWS_PER_W

    def softplus_slot(slot):
        def body(r, carry):
            for k in range(DIM // LANES):
                sl = pl.ds(k * LANES, LANES)
                bufv[slot, r, sl] = _softplus16(bufv[slot, r, sl])
            return carry
        lax.fori_loop(0, CHUNK, body, 0, unroll=2)

    def start_gather(c):
        slot = c % 2
        cu = pltpu.async_copy(u_hbm.at[idx_v.at[c]], bufu.at[slot], gsem)
        cv = pltpu.async_copy(v_hbm.at[idx_v.at[c]], bufv.at[slot], gsem)
        return cu, cv

    def start_out(c):
        slot = c % 2
        rows = pl.ds(base + c * CHUNK, CHUNK)
        ou = pltpu.async_copy(bufu.at[slot], out_hbm.at[rows, pl.ds(0, DIM)],
                              osem)
        ov = pltpu.async_copy(bufv.at[slot], out_hbm.at[rows, pl.ds(DIM, DIM)],
                              osem)
        return ou, ov

    gathers = {0: start_gather(0)}
    outs = {}
    for c in range(NCHUNK):
        gu, gv = gathers.pop(c)
        gu.wait()
        gv.wait()
        if c + 1 < NCHUNK:
            if c - 1 in outs:
                ou, ov = outs.pop(c - 1)
                ou.wait()
                ov.wait()
            gathers[c + 1] = start_gather(c + 1)
        softplus_slot(c % 2)
        outs[c] = start_out(c)
    ou, ov = outs.pop(NCHUNK - 1)
    ou.wait()
    ov.wait()


@jax.jit
def _hard_box_sc(idx3, u, v):
    mesh = plsc.VectorSubcoreMesh(core_axis_name="c", subcore_axis_name="s")
    k = functools.partial(
        pl.kernel,
        out_type=jax.ShapeDtypeStruct((BATCH2, 2 * DIM), jnp.float32),
        mesh=mesh,
        scratch_types=[
            pltpu.VMEM((NCHUNK, CHUNK), jnp.int32),
            pltpu.VMEM((2, CHUNK, DIM), jnp.float32),
            pltpu.VMEM((2, CHUNK, DIM), jnp.float32),
            pltpu.SemaphoreType.DMA,
            pltpu.SemaphoreType.DMA,
        ],
        compiler_params=pltpu.CompilerParams(use_tc_tiling_on_sc=False),
    )(_sc_body)
    return k(idx3, u, v)


def kernel(idxs, U, V):
    idx3 = idxs.reshape(-1).astype(jnp.int32).reshape(NW, NCHUNK, CHUNK)
    out = _hard_box_sc(idx3, U, V)
    return out.reshape(idxs.shape[0], 2, 2, DIM)
